# Initial kernel scaffold; baseline (speedup 1.0000x reference)
import jax
import jax.numpy as jnp
from jax.experimental import pallas as pl


def kernel(x, Wr, We, Ws):
    raise NotImplementedError("write your pallas kernel here")


# speedup vs baseline: 1.5269x; 1.5269x over previous
import jax
import jax.numpy as jnp
from jax.experimental import pallas as pl
from jax.experimental.pallas import tpu as pltpu

_B, _C1, _C2, _H, _W = 4, 384, 384, 56, 56
_E, _K = 8, 2
_HW = _H * _W


def _moe_kernel(x_ref, Wr_ref, We_ref, Ws_ref, out_ref):
    xb = x_ref[0]
    gap = jnp.mean(xb, axis=1, keepdims=True)
    logits = jnp.sum(gap * Wr_ref[...], axis=0, keepdims=True)
    iota = jax.lax.broadcasted_iota(jnp.int32, (1, _E), 1)
    m1 = jnp.max(logits)
    i1 = jnp.min(jnp.where(logits == m1, iota, _E))
    masked = jnp.where(iota == i1, -jnp.inf, logits)
    m2 = jnp.max(masked)
    i2 = jnp.min(jnp.where(masked == m2, iota, _E))
    e = jnp.exp(m2 - m1)
    w0 = 1.0 / (1.0 + e)
    w1 = e / (1.0 + e)
    Wc = w0 * We_ref[i1] + w1 * We_ref[i2] + Ws_ref[...]
    out_ref[0] = jnp.dot(Wc, xb, preferred_element_type=jnp.float32) + xb


def kernel(x, Wr, We, Ws):
    xr = x.reshape(_B, _C1, _HW)
    out = pl.pallas_call(
        _moe_kernel,
        grid=(_B,),
        in_specs=[
            pl.BlockSpec((1, _C1, _HW), lambda b: (b, 0, 0)),
            pl.BlockSpec((_C1, _E), lambda b: (0, 0)),
            pl.BlockSpec((_E, _C2, _C1), lambda b: (0, 0, 0)),
            pl.BlockSpec((_C2, _C1), lambda b: (0, 0)),
        ],
        out_specs=pl.BlockSpec((1, _C2, _HW), lambda b: (b, 0, 0)),
        out_shape=jax.ShapeDtypeStruct((_B, _C2, _HW), jnp.float32),
        compiler_params=pltpu.CompilerParams(
            dimension_semantics=("arbitrary",),
        ),
    )(xr, Wr, We, Ws)
    return out.reshape(_B, _C2, _H, _W)
